# Initial kernel scaffold; baseline (speedup 1.0000x reference)
import jax
import jax.numpy as jnp
from jax.experimental import pallas as pl


def kernel(x_nchw, conv1_w, conv1_b, conv2_w, conv2_b, fc1_w, fc1_b, fc2_w, fc2_b, fc3_w, fc3_b):
    raise NotImplementedError("write your pallas kernel here")


# speedup vs baseline: 34.6114x; 34.6114x over previous
import numpy as np
import jax
import jax.numpy as jnp
from jax.experimental import pallas as pl
from jax.experimental.pallas import tpu as pltpu


def _lenet_kernel(x_ref, w1_ref, b1_ref, w2_ref, b2_ref,
                  wf1_ref, bf1_ref, wf2_ref, bf2_ref, wf3_ref, bf3_ref,
                  o_ref):
    tb = x_ref.shape[0]
    xf = x_ref[...].reshape(tb * 8, 384)

    y = None
    for s in range(2):
        p = jnp.dot(xf, w1_ref[s], preferred_element_type=jnp.float32)
        p = p.reshape(tb, 8, 1024)[:, s:s + 7, :]
        y = p if y is None else y + p
    y = jnp.maximum(y + b1_ref[...].reshape(1, 1, 1024), 0.0)
    ja = jnp.maximum(jnp.maximum(y[:, :, 0:128], y[:, :, 128:256]),
                     jnp.maximum(y[:, :, 256:384], y[:, :, 384:512]))
    jb = jnp.maximum(jnp.maximum(y[:, :, 512:640], y[:, :, 640:768]),
                     jnp.maximum(y[:, :, 768:896], y[:, :, 896:1024]))
    a1 = jnp.concatenate([ja, jb], axis=2)
    a1 = jnp.concatenate([a1, jnp.zeros((tb, 1, 256), a1.dtype)], axis=1)
    a1f = a1.reshape(tb * 8, 256)

    y2 = None
    for s in range(3):
        p = jnp.dot(a1f, w2_ref[s], preferred_element_type=jnp.float32)
        p = p.reshape(tb, 8, 512)[:, s:s + 5, :]
        y2 = p if y2 is None else y2 + p
    y2 = jnp.maximum(y2 + b2_ref[...].reshape(1, 1, 512), 0.0)
    a2 = jnp.maximum(jnp.maximum(y2[:, :, 0:128], y2[:, :, 128:256]),
                     jnp.maximum(y2[:, :, 256:384], y2[:, :, 384:512]))

    h = None
    for i in range(5):
        q = jnp.dot(a2[:, i, :], wf1_ref[i], preferred_element_type=jnp.float32)
        h = q if h is None else h + q
    h = jnp.maximum(h + bf1_ref[...], 0.0)
    h = jnp.maximum(jnp.dot(h, wf2_ref[...], preferred_element_type=jnp.float32)
                    + bf2_ref[...], 0.0)
    o_ref[...] = (jnp.dot(h, wf3_ref[...], preferred_element_type=jnp.float32)
                  + bf3_ref[...])


def _conv1_toeplitz(conv1_w):
    wt = jnp.transpose(conv1_w, (2, 3, 1, 0))
    s_ = np.arange(2)[:, None, None, None]
    r_ = np.arange(4)[None, :, None, None]
    jp_ = np.arange(2)[None, None, :, None]
    p_ = np.arange(2)[None, None, None, :]
    kh = 4 * s_ + r_ - 2 * jp_ - p_
    wq_ = np.arange(32)[:, None, None]
    wp_ = np.arange(2)[None, :, None]
    wh_ = np.arange(14)[None, None, :]
    kw = wq_ - 2 * wh_ - wp_
    vkh = (kh >= 0) & (kh <= 4)
    vkw = (kw >= 0) & (kw <= 4)
    full = (2, 4, 2, 2, 32, 2, 14)
    KH = np.broadcast_to(np.clip(kh, 0, 4)[:, :, :, :, None, None, None], full)
    KW = np.broadcast_to(np.clip(kw, 0, 4)[None, None, None, None], full)
    mask = vkh[:, :, :, :, None, None, None] & vkw[None, None, None, None]
    g = wt[KH, KW]
    g = g * jnp.asarray(mask, g.dtype)[..., None, None]
    g = jnp.transpose(g, (0, 1, 7, 4, 2, 3, 5, 6, 8))
    g = g.reshape(2, 384, 8, 84)
    g = jnp.pad(g, ((0, 0), (0, 0), (0, 0), (0, 44)))
    return g.reshape(2, 384, 1024)


def _conv2_toeplitz(conv2_w):
    wt = jnp.transpose(conv2_w, (2, 3, 1, 0))
    s_ = np.arange(3)[:, None, None]
    r_ = np.arange(2)[None, :, None]
    p2_ = np.arange(2)[None, None, :]
    kh = 2 * s_ + r_ - p2_
    wq_ = np.arange(14)[:, None, None]
    wp_ = np.arange(2)[None, :, None]
    wh_ = np.arange(5)[None, None, :]
    kw = wq_ - 2 * wh_ - wp_
    vkh = (kh >= 0) & (kh <= 4)
    vkw = (kw >= 0) & (kw <= 4)
    full = (3, 2, 2, 14, 2, 5)
    KH = np.broadcast_to(np.clip(kh, 0, 4)[:, :, :, None, None, None], full)
    KW = np.broadcast_to(np.clip(kw, 0, 4)[None, None, None], full)
    mask = vkh[:, :, :, None, None, None] & vkw[None, None, None]
    g = wt[KH, KW]
    g = g * jnp.asarray(mask, g.dtype)[..., None, None]
    g = jnp.transpose(g, (0, 1, 3, 6, 2, 4, 5, 7))
    g = g.reshape(3, 2, 84, 4, 80)
    g = jnp.pad(g, ((0, 0), (0, 0), (0, 44), (0, 0), (0, 48)))
    return g.reshape(3, 256, 512)


def kernel(x_nchw, conv1_w, conv1_b, conv2_w, conv2_b,
           fc1_w, fc1_b, fc2_w, fc2_b, fc3_w, fc3_b):
    B = x_nchw.shape[0]
    x = jnp.transpose(x_nchw, (0, 2, 1, 3)).reshape(B, 8, 384)

    w1 = _conv1_toeplitz(conv1_w)
    bh = jnp.pad(jnp.tile(conv1_b, 14), (0, 44))
    b1 = jnp.tile(bh, 8).reshape(1, 1024)
    w2 = _conv2_toeplitz(conv2_w)
    bh2 = jnp.pad(jnp.tile(conv2_b, 5), (0, 48))
    b2 = jnp.tile(bh2, 4).reshape(1, 512)

    f = jnp.transpose(fc1_w.reshape(16, 5, 5, 120), (1, 2, 0, 3))
    wf1 = jnp.pad(f.reshape(5, 80, 120), ((0, 0), (0, 48), (0, 8)))
    bf1 = jnp.pad(fc1_b, (0, 8)).reshape(1, 128)
    wf2 = jnp.pad(fc2_w, ((0, 8), (0, 44)))
    bf2 = jnp.pad(fc2_b, (0, 44)).reshape(1, 128)
    wf3 = jnp.pad(fc3_w, ((0, 44), (0, 118)))
    bf3 = jnp.pad(fc3_b, (0, 118)).reshape(1, 128)

    TB = 128
    while B % TB:
        TB //= 2

    out = pl.pallas_call(
        _lenet_kernel,
        out_shape=jax.ShapeDtypeStruct((B, 128), jnp.float32),
        grid=(B // TB,),
        in_specs=[
            pl.BlockSpec((TB, 8, 384), lambda i: (i, 0, 0)),
            pl.BlockSpec((2, 384, 1024), lambda i: (0, 0, 0)),
            pl.BlockSpec((1, 1024), lambda i: (0, 0)),
            pl.BlockSpec((3, 256, 512), lambda i: (0, 0, 0)),
            pl.BlockSpec((1, 512), lambda i: (0, 0)),
            pl.BlockSpec((5, 128, 128), lambda i: (0, 0, 0)),
            pl.BlockSpec((1, 128), lambda i: (0, 0)),
            pl.BlockSpec((128, 128), lambda i: (0, 0)),
            pl.BlockSpec((1, 128), lambda i: (0, 0)),
            pl.BlockSpec((128, 128), lambda i: (0, 0)),
            pl.BlockSpec((1, 128), lambda i: (0, 0)),
        ],
        out_specs=pl.BlockSpec((TB, 128), lambda i: (i, 0)),
        compiler_params=pltpu.CompilerParams(dimension_semantics=("parallel",)),
    )(x, w1, b1, w2, b2, wf1, bf1, wf2, bf2, wf3, bf3)
    return out[:, :10]


# speedup vs baseline: 41.2031x; 1.1904x over previous
import numpy as np
import jax
import jax.numpy as jnp
from jax.experimental import pallas as pl
from jax.experimental.pallas import tpu as pltpu


def _lenet_kernel(x_ref, w1_ref, b1_ref, w2_ref, b2_ref,
                  wf1_ref, bf1_ref, wf2_ref, bf2_ref, wf3_ref, bf3_ref,
                  o_ref):
    tb = x_ref.shape[0]
    x = x_ref[...]
    xf = jnp.concatenate([x[:, 0], x[:, 1], x[:, 2]], axis=2).reshape(tb * 8, 384)

    y = None
    for s in range(2):
        p = jnp.dot(xf, w1_ref[s], preferred_element_type=jnp.float32)
        p = p.reshape(tb, 8, 1024)[:, s:s + 7, :]
        y = p if y is None else y + p
    y = jnp.maximum(y + b1_ref[...].reshape(1, 1, 1024), 0.0)
    ja = jnp.maximum(jnp.maximum(y[:, :, 0:128], y[:, :, 128:256]),
                     jnp.maximum(y[:, :, 256:384], y[:, :, 384:512]))
    jb = jnp.maximum(jnp.maximum(y[:, :, 512:640], y[:, :, 640:768]),
                     jnp.maximum(y[:, :, 768:896], y[:, :, 896:1024]))
    a1 = jnp.concatenate([ja, jb], axis=2)
    a1 = jnp.concatenate([a1, jnp.zeros((tb, 1, 256), a1.dtype)], axis=1)
    a1f = a1.reshape(tb * 8, 256)

    y2 = None
    for s in range(3):
        p = jnp.dot(a1f, w2_ref[s], preferred_element_type=jnp.float32)
        p = p.reshape(tb, 8, 512)[:, s:s + 5, :]
        y2 = p if y2 is None else y2 + p
    y2 = jnp.maximum(y2 + b2_ref[...].reshape(1, 1, 512), 0.0)
    a2 = jnp.maximum(jnp.maximum(y2[:, :, 0:128], y2[:, :, 128:256]),
                     jnp.maximum(y2[:, :, 256:384], y2[:, :, 384:512]))

    h = None
    for i in range(5):
        q = jnp.dot(a2[:, i, :], wf1_ref[i], preferred_element_type=jnp.float32)
        h = q if h is None else h + q
    h = jnp.maximum(h + bf1_ref[...], 0.0)
    h = jnp.maximum(jnp.dot(h, wf2_ref[...], preferred_element_type=jnp.float32)
                    + bf2_ref[...], 0.0)
    o_ref[...] = (jnp.dot(h, wf3_ref[...], preferred_element_type=jnp.float32)
                  + bf3_ref[...])


def _conv1_toeplitz(conv1_w):
    wt = jnp.transpose(conv1_w, (2, 3, 1, 0))
    s_ = np.arange(2)[:, None, None, None]
    r_ = np.arange(4)[None, :, None, None]
    jp_ = np.arange(2)[None, None, :, None]
    p_ = np.arange(2)[None, None, None, :]
    kh = 4 * s_ + r_ - 2 * jp_ - p_
    wq_ = np.arange(32)[:, None, None]
    wp_ = np.arange(2)[None, :, None]
    wh_ = np.arange(14)[None, None, :]
    kw = wq_ - 2 * wh_ - wp_
    vkh = (kh >= 0) & (kh <= 4)
    vkw = (kw >= 0) & (kw <= 4)
    full = (2, 4, 2, 2, 32, 2, 14)
    KH = np.broadcast_to(np.clip(kh, 0, 4)[:, :, :, :, None, None, None], full)
    KW = np.broadcast_to(np.clip(kw, 0, 4)[None, None, None, None], full)
    mask = vkh[:, :, :, :, None, None, None] & vkw[None, None, None, None]
    g = wt[KH, KW]
    g = g * jnp.asarray(mask, g.dtype)[..., None, None]
    g = jnp.transpose(g, (0, 7, 1, 4, 2, 3, 5, 6, 8))
    g = g.reshape(2, 384, 8, 84)
    g = jnp.pad(g, ((0, 0), (0, 0), (0, 0), (0, 44)))
    return g.reshape(2, 384, 1024)


def _conv2_toeplitz(conv2_w):
    wt = jnp.transpose(conv2_w, (2, 3, 1, 0))
    s_ = np.arange(3)[:, None, None]
    r_ = np.arange(2)[None, :, None]
    p2_ = np.arange(2)[None, None, :]
    kh = 2 * s_ + r_ - p2_
    wq_ = np.arange(14)[:, None, None]
    wp_ = np.arange(2)[None, :, None]
    wh_ = np.arange(5)[None, None, :]
    kw = wq_ - 2 * wh_ - wp_
    vkh = (kh >= 0) & (kh <= 4)
    vkw = (kw >= 0) & (kw <= 4)
    full = (3, 2, 2, 14, 2, 5)
    KH = np.broadcast_to(np.clip(kh, 0, 4)[:, :, :, None, None, None], full)
    KW = np.broadcast_to(np.clip(kw, 0, 4)[None, None, None], full)
    mask = vkh[:, :, :, None, None, None] & vkw[None, None, None]
    g = wt[KH, KW]
    g = g * jnp.asarray(mask, g.dtype)[..., None, None]
    g = jnp.transpose(g, (0, 1, 3, 6, 2, 4, 5, 7))
    g = g.reshape(3, 2, 84, 4, 80)
    g = jnp.pad(g, ((0, 0), (0, 0), (0, 44), (0, 0), (0, 48)))
    return g.reshape(3, 256, 512)


def kernel(x_nchw, conv1_w, conv1_b, conv2_w, conv2_b,
           fc1_w, fc1_b, fc2_w, fc2_b, fc3_w, fc3_b):
    B = x_nchw.shape[0]
    x = x_nchw.reshape(B, 3, 8, 128)

    w1 = _conv1_toeplitz(conv1_w)
    bh = jnp.pad(jnp.tile(conv1_b, 14), (0, 44))
    b1 = jnp.tile(bh, 8).reshape(1, 1024)
    w2 = _conv2_toeplitz(conv2_w)
    bh2 = jnp.pad(jnp.tile(conv2_b, 5), (0, 48))
    b2 = jnp.tile(bh2, 4).reshape(1, 512)

    f = jnp.transpose(fc1_w.reshape(16, 5, 5, 120), (1, 2, 0, 3))
    wf1 = jnp.pad(f.reshape(5, 80, 120), ((0, 0), (0, 48), (0, 8)))
    bf1 = jnp.pad(fc1_b, (0, 8)).reshape(1, 128)
    wf2 = jnp.pad(fc2_w, ((0, 8), (0, 44)))
    bf2 = jnp.pad(fc2_b, (0, 44)).reshape(1, 128)
    wf3 = jnp.pad(fc3_w, ((0, 44), (0, 118)))
    bf3 = jnp.pad(fc3_b, (0, 118)).reshape(1, 128)

    TB = 128
    while B % TB:
        TB //= 2

    out = pl.pallas_call(
        _lenet_kernel,
        out_shape=jax.ShapeDtypeStruct((B, 128), jnp.float32),
        grid=(B // TB,),
        in_specs=[
            pl.BlockSpec((TB, 3, 8, 128), lambda i: (i, 0, 0, 0)),
            pl.BlockSpec((2, 384, 1024), lambda i: (0, 0, 0)),
            pl.BlockSpec((1, 1024), lambda i: (0, 0)),
            pl.BlockSpec((3, 256, 512), lambda i: (0, 0, 0)),
            pl.BlockSpec((1, 512), lambda i: (0, 0)),
            pl.BlockSpec((5, 128, 128), lambda i: (0, 0, 0)),
            pl.BlockSpec((1, 128), lambda i: (0, 0)),
            pl.BlockSpec((128, 128), lambda i: (0, 0)),
            pl.BlockSpec((1, 128), lambda i: (0, 0)),
            pl.BlockSpec((128, 128), lambda i: (0, 0)),
            pl.BlockSpec((1, 128), lambda i: (0, 0)),
        ],
        out_specs=pl.BlockSpec((TB, 128), lambda i: (i, 0)),
        compiler_params=pltpu.CompilerParams(dimension_semantics=("parallel",)),
    )(x, w1, b1, w2, b2, wf1, bf1, wf2, bf2, wf3, bf3)
    return out[:, :10]


# speedup vs baseline: 117.5213x; 2.8522x over previous
import numpy as np
import jax
import jax.numpy as jnp
from jax.experimental import pallas as pl
from jax.experimental.pallas import tpu as pltpu


def _lenet_kernel(x_ref, w1_ref, b1_ref, w2_ref, b2_ref,
                  wf1_ref, bf1_ref, wf2_ref, bf2_ref, wf3_ref, bf3_ref,
                  o_ref):
    tb = x_ref.shape[0]
    x = x_ref[...]
    xf = jnp.concatenate([x[:, 0], x[:, 1], x[:, 2]], axis=2).reshape(tb * 8, 384)

    y = None
    for s in range(2):
        p = jnp.dot(xf, w1_ref[s], preferred_element_type=jnp.float32)
        p = p.reshape(tb, 8, 1024)[:, s:s + 7, :]
        y = p if y is None else y + p
    y = jnp.maximum(y + b1_ref[...].reshape(1, 1, 1024), 0.0)
    ja = jnp.maximum(jnp.maximum(y[:, :, 0:128], y[:, :, 128:256]),
                     jnp.maximum(y[:, :, 256:384], y[:, :, 384:512]))
    jb = jnp.maximum(jnp.maximum(y[:, :, 512:640], y[:, :, 640:768]),
                     jnp.maximum(y[:, :, 768:896], y[:, :, 896:1024]))
    a1 = jnp.concatenate([ja, jb], axis=2)
    a1 = jnp.concatenate([a1, jnp.zeros((tb, 1, 256), a1.dtype)], axis=1)
    a1f = a1.reshape(tb * 8, 256)

    y2 = None
    for s in range(3):
        p = jnp.dot(a1f, w2_ref[s], preferred_element_type=jnp.float32)
        p = p.reshape(tb, 8, 512)[:, s:s + 5, :]
        y2 = p if y2 is None else y2 + p
    y2 = jnp.maximum(y2 + b2_ref[...].reshape(1, 1, 512), 0.0)
    a2 = jnp.maximum(jnp.maximum(y2[:, :, 0:128], y2[:, :, 128:256]),
                     jnp.maximum(y2[:, :, 256:384], y2[:, :, 384:512]))

    h = None
    for i in range(5):
        q = jnp.dot(a2[:, i, :], wf1_ref[i], preferred_element_type=jnp.float32)
        h = q if h is None else h + q
    h = jnp.maximum(h + bf1_ref[...], 0.0)
    h = jnp.maximum(jnp.dot(h, wf2_ref[...], preferred_element_type=jnp.float32)
                    + bf2_ref[...], 0.0)
    o_ref[...] = (jnp.dot(h, wf3_ref[...], preferred_element_type=jnp.float32)
                  + bf3_ref[...])


def _onehot(idx, valid, depth):
    oh = np.zeros(idx.shape + (depth,), np.float32)
    np.put_along_axis(oh, np.clip(idx, 0, depth - 1)[..., None], 1.0, axis=-1)
    return jnp.asarray(oh * valid[..., None])


def _conv1_toeplitz(conv1_w):
    s_ = np.arange(2)[:, None, None, None]
    r_ = np.arange(4)[None, :, None, None]
    jp_ = np.arange(2)[None, None, :, None]
    p_ = np.arange(2)[None, None, None, :]
    kh = 4 * s_ + r_ - 2 * jp_ - p_
    wq_ = np.arange(32)[:, None, None]
    wp_ = np.arange(2)[None, :, None]
    wh_ = np.arange(14)[None, None, :]
    kw = wq_ - 2 * wh_ - wp_
    E = _onehot(kh, (kh >= 0) & (kh <= 4), 5)
    F = _onehot(kw, (kw >= 0) & (kw <= 4), 5)
    cwt = jnp.transpose(conv1_w, (1, 2, 3, 0))
    g = jnp.einsum('srjpk,wvhl,cklo->scrwjpvho', E, F, cwt)
    g = g.reshape(2, 384, 8, 84)
    g = jnp.pad(g, ((0, 0), (0, 0), (0, 0), (0, 44)))
    return g.reshape(2, 384, 1024)


def _conv2_toeplitz(conv2_w):
    s_ = np.arange(3)[:, None, None]
    r_ = np.arange(2)[None, :, None]
    p2_ = np.arange(2)[None, None, :]
    kh = 2 * s_ + r_ - p2_
    wq_ = np.arange(14)[:, None, None]
    wp_ = np.arange(2)[None, :, None]
    wh_ = np.arange(5)[None, None, :]
    kw = wq_ - 2 * wh_ - wp_
    E = _onehot(kh, (kh >= 0) & (kh <= 4), 5)
    F = _onehot(kw, (kw >= 0) & (kw <= 4), 5)
    cwt = jnp.transpose(conv2_w, (1, 2, 3, 0))
    g = jnp.einsum('srpk,wvhl,cklo->srwcpvho', E, F, cwt)
    g = g.reshape(3, 2, 84, 4, 80)
    g = jnp.pad(g, ((0, 0), (0, 0), (0, 44), (0, 0), (0, 48)))
    return g.reshape(3, 256, 512)


def kernel(x_nchw, conv1_w, conv1_b, conv2_w, conv2_b,
           fc1_w, fc1_b, fc2_w, fc2_b, fc3_w, fc3_b):
    B = x_nchw.shape[0]
    x = x_nchw.reshape(B, 3, 8, 128)

    w1 = _conv1_toeplitz(conv1_w)
    bh = jnp.pad(jnp.tile(conv1_b, 14), (0, 44))
    b1 = jnp.tile(bh, 8).reshape(1, 1024)
    w2 = _conv2_toeplitz(conv2_w)
    bh2 = jnp.pad(jnp.tile(conv2_b, 5), (0, 48))
    b2 = jnp.tile(bh2, 4).reshape(1, 512)

    f = jnp.transpose(fc1_w.reshape(16, 5, 5, 120), (1, 2, 0, 3))
    wf1 = jnp.pad(f.reshape(5, 80, 120), ((0, 0), (0, 48), (0, 8)))
    bf1 = jnp.pad(fc1_b, (0, 8)).reshape(1, 128)
    wf2 = jnp.pad(fc2_w, ((0, 8), (0, 44)))
    bf2 = jnp.pad(fc2_b, (0, 44)).reshape(1, 128)
    wf3 = jnp.pad(fc3_w, ((0, 44), (0, 118)))
    bf3 = jnp.pad(fc3_b, (0, 118)).reshape(1, 128)

    TB = 128
    while B % TB:
        TB //= 2

    out = pl.pallas_call(
        _lenet_kernel,
        out_shape=jax.ShapeDtypeStruct((B, 128), jnp.float32),
        grid=(B // TB,),
        in_specs=[
            pl.BlockSpec((TB, 3, 8, 128), lambda i: (i, 0, 0, 0)),
            pl.BlockSpec((2, 384, 1024), lambda i: (0, 0, 0)),
            pl.BlockSpec((1, 1024), lambda i: (0, 0)),
            pl.BlockSpec((3, 256, 512), lambda i: (0, 0, 0)),
            pl.BlockSpec((1, 512), lambda i: (0, 0)),
            pl.BlockSpec((5, 128, 128), lambda i: (0, 0, 0)),
            pl.BlockSpec((1, 128), lambda i: (0, 0)),
            pl.BlockSpec((128, 128), lambda i: (0, 0)),
            pl.BlockSpec((1, 128), lambda i: (0, 0)),
            pl.BlockSpec((128, 128), lambda i: (0, 0)),
            pl.BlockSpec((1, 128), lambda i: (0, 0)),
        ],
        out_specs=pl.BlockSpec((TB, 128), lambda i: (i, 0)),
        compiler_params=pltpu.CompilerParams(dimension_semantics=("parallel",)),
    )(x, w1, b1, w2, b2, wf1, bf1, wf2, bf2, wf3, bf3)
    return out[:, :10]
